# Initial kernel scaffold; baseline (speedup 1.0000x reference)
import jax
import jax.numpy as jnp
from jax.experimental import pallas as pl


def kernel(hidden, batch_ngram_bucket_ids, tables, Wk, Wv, qn_w, kn_w, vn_w, conv_w, conv_b):
    raise NotImplementedError("write your pallas kernel here")


# speedup vs baseline: 1.8539x; 1.8539x over previous
import functools
import math

import jax
import jax.numpy as jnp
from jax import lax
from jax.experimental import pallas as pl
from jax.experimental.pallas import tpu as pltpu
from jax.experimental.pallas import tpu_sc as plsc

HIDDEN = 1024
MEM = 512
BUCKETS = 100000
SLOTS = 8
SLOT_DIM = MEM // SLOTS
N = 16384

NC = 2
NS = 16
NW = NC * NS
TOTAL_ROWS = N * SLOTS
ROWS_PER_W = TOTAL_ROWS // NW
CHUNK = 128
CHUNKS_PER_W = ROWS_PER_W // CHUNK


def _sc_gather(table_hbm, idx_hbm, out_hbm, idx_v, rows_v, sem):
    wid = lax.axis_index("s") * NC + lax.axis_index("c")
    base = wid * ROWS_PER_W
    pltpu.sync_copy(idx_hbm.at[wid], idx_v)

    def body(j, carry):
        pltpu.async_copy(table_hbm.at[idx_v.at[j]], rows_v, sem).wait()
        pltpu.sync_copy(rows_v, out_hbm.at[pl.ds(base + j * CHUNK, CHUNK)])
        return carry

    lax.fori_loop(0, CHUNKS_PER_W, body, 0)


def _make_gather_call():
    return functools.partial(
        pl.kernel,
        out_type=jax.ShapeDtypeStruct((TOTAL_ROWS, SLOT_DIM), jnp.float32),
        mesh=plsc.VectorSubcoreMesh(core_axis_name="c", subcore_axis_name="s",
                                    num_cores=NC, num_subcores=NS),
        scratch_types=[
            pltpu.VMEM((CHUNKS_PER_W, CHUNK), jnp.int32),
            pltpu.VMEM((CHUNK, SLOT_DIM), jnp.float32),
            pltpu.SemaphoreType.DMA,
        ],
        compiler_params=pltpu.CompilerParams(use_tc_tiling_on_sc=False),
    )(_sc_gather)


def _dense_body(hid_ref, mem_ref, wkt_ref, wvt_ref, qn_ref, kn_ref, vn_ref,
                cw_ref, cb_ref, out_ref):
    eps = 1e-8
    q = hid_ref[...]
    q = q * lax.rsqrt(jnp.mean(q * q, axis=-1, keepdims=True) + eps)
    q = q * qn_ref[...]
    m = mem_ref[...]
    k = jnp.dot(m, wkt_ref[...], preferred_element_type=jnp.float32)
    k = k * lax.rsqrt(jnp.mean(k * k, axis=-1, keepdims=True) + eps)
    k = k * kn_ref[...]
    v = jnp.dot(m, wvt_ref[...], preferred_element_type=jnp.float32)
    v = v * lax.rsqrt(jnp.mean(v * v, axis=-1, keepdims=True) + eps)
    v = v * vn_ref[...]
    logits = jnp.sum(q * k, axis=-1, keepdims=True) * (1.0 / math.sqrt(HIDDEN))
    alpha = jax.nn.sigmoid(logits)
    g = alpha * v
    co = g * cw_ref[...] + cb_ref[...]
    out_ref[...] = co * jax.nn.sigmoid(co) + g


def kernel(hidden, batch_ngram_bucket_ids, tables, Wk, Wv, qn_w, kn_w, vn_w,
           conv_w, conv_b):
    ids = jnp.asarray(batch_ngram_bucket_ids, jnp.int32)
    flat_ids = ids + (jnp.arange(SLOTS, dtype=jnp.int32) * BUCKETS)[None, :]
    idx = flat_ids.reshape(NW, CHUNKS_PER_W, CHUNK)
    flat_tables = tables.reshape(SLOTS * BUCKETS, SLOT_DIM)

    rows = _make_gather_call()(flat_tables, idx)
    memory = rows.reshape(N, MEM)

    bn = 1024
    grid = (N // bn,)
    full = lambda i: (0, 0)
    vec = lambda x: x.reshape(1, HIDDEN)
    out = pl.pallas_call(
        _dense_body,
        grid=grid,
        in_specs=[
            pl.BlockSpec((bn, HIDDEN), lambda i: (i, 0)),
            pl.BlockSpec((bn, MEM), lambda i: (i, 0)),
            pl.BlockSpec((MEM, HIDDEN), full),
            pl.BlockSpec((MEM, HIDDEN), full),
            pl.BlockSpec((1, HIDDEN), full),
            pl.BlockSpec((1, HIDDEN), full),
            pl.BlockSpec((1, HIDDEN), full),
            pl.BlockSpec((1, HIDDEN), full),
            pl.BlockSpec((1, HIDDEN), full),
        ],
        out_specs=pl.BlockSpec((bn, HIDDEN), lambda i: (i, 0)),
        out_shape=jax.ShapeDtypeStruct((N, HIDDEN), jnp.float32),
    )(hidden, memory, Wk.T, Wv.T, vec(qn_w), vec(kn_w), vec(vn_w),
      vec(conv_w[:, 0, 2]), vec(conv_b))
    return out
